# baseline (speedup 1.0000x reference)
import functools

import jax
import jax.numpy as jnp
from jax import lax
from jax.experimental import pallas as pl
from jax.experimental.pallas import tpu as pltpu
from jax.experimental.pallas import tpu_sc as plsc

N = 10000
E = 320000
FIN = 128
FOUT = 32

NC = 2
NS = 16
NW = NC * NS
CHUNK = 128
NCHUNK = 79
EPW = NCHUNK * CHUNK
EPAD = EPW * NW

DUMP = 10008
SP_H = 10016
SP_Z = NCHUNK * CHUNK
ZROWS_PER_TILE = SP_Z // NS
SP_D = 10240

_mesh = plsc.VectorSubcoreMesh(core_axis_name="c", subcore_axis_name="s")


def _zeros16():
    return jnp.zeros((16,), jnp.float32)


def _ones16():
    return jnp.ones((16,), jnp.float32)


@functools.partial(
    pl.kernel,
    out_type=[
        jax.ShapeDtypeStruct((NW, SP_D), jnp.float32),
        jax.ShapeDtypeStruct((NW, SP_D), jnp.float32),
    ],
    mesh=_mesh,
    compiler_params=pltpu.CompilerParams(needs_layout_passes=False),
    scratch_types=[
        pltpu.VMEM((NCHUNK, CHUNK), jnp.int32),
        pltpu.VMEM((NCHUNK, CHUNK), jnp.int32),
        pltpu.VMEM((SP_D,), jnp.float32),
        pltpu.VMEM((SP_D,), jnp.float32),
    ],
)
def _deg_kernel(src_hbm, dst_hbm, degs_out, degd_out, ids_v, idd_v, hls, hld):
    cid = lax.axis_index("c")
    sid = lax.axis_index("s")
    w = cid * NS + sid

    pltpu.sync_copy(src_hbm.at[w], ids_v)
    pltpu.sync_copy(dst_hbm.at[w], idd_v)

    def zero_row(i, _):
        z = _zeros16()
        hls[pl.ds(i * 16, 16)] = z
        hld[pl.ds(i * 16, 16)] = z
        return 0
    lax.fori_loop(0, SP_D // 16, zero_row, 0)

    def accum(i, _):
        vs = ids_v[i // 8, pl.ds((i % 8) * 16, 16)]
        plsc.addupdate_scatter(hls, [vs], _ones16())
        vd = idd_v[i // 8, pl.ds((i % 8) * 16, 16)]
        plsc.addupdate_scatter(hld, [vd], _ones16())
        return 0
    lax.fori_loop(0, EPW // 16, accum, 0)

    pltpu.sync_copy(hls, degs_out.at[w])
    pltpu.sync_copy(hld, degd_out.at[w])


@functools.partial(
    pl.kernel,
    out_type=jax.ShapeDtypeStruct((NC, SP_Z, FOUT), jnp.float32),
    mesh=_mesh,
    compiler_params=pltpu.CompilerParams(use_tc_tiling_on_sc=False),
    scratch_types=[
        pltpu.VMEM((NCHUNK, CHUNK), jnp.int32),
        pltpu.VMEM((NCHUNK, CHUNK), jnp.int32),
        [pltpu.VMEM((CHUNK, FOUT), jnp.float32)] * 4,
        pltpu.VMEM((CHUNK, FOUT), jnp.float32),
        pltpu.VMEM_SHARED((SP_Z, FOUT), jnp.float32),
        [pltpu.SemaphoreType.DMA] * 4,
        [pltpu.SemaphoreType.DMA] * 4,
    ],
)
def _msg_kernel(h_hbm, src_hbm, dst_hbm, zp_out,
                ids_v, idd_v, bufs, zblk, shz, semg, sems):
    cid = lax.axis_index("c")
    sid = lax.axis_index("s")
    w = cid * NS + sid

    pltpu.sync_copy(src_hbm.at[w], ids_v)
    pltpu.sync_copy(dst_hbm.at[w], idd_v)

    def zero_row(i, _):
        z = _zeros16()
        zblk[i, pl.ds(0, 16)] = z
        zblk[i, pl.ds(16, 16)] = z
        return 0
    lax.fori_loop(0, CHUNK, zero_row, 0)

    for k in range((NCHUNK + NS - 1) // NS):
        j = sid + k * NS

        @pl.when(j < NCHUNK)
        def _():
            pltpu.sync_copy(zblk, shz.at[pl.ds(j * CHUNK, CHUNK)])

    plsc.subcore_barrier()

    pltpu.async_copy(h_hbm.at[ids_v.at[0]], bufs[0], semg[0])

    def edge_group(g, _):
        for u in range(4):
            k = 4 * g + u
            bn = (u + 1) % 4
            kn = k + 1

            @pl.when(kn < NCHUNK)
            def _():
                @pl.when(k - 3 >= 0)
                def _():
                    pltpu.make_async_copy(
                        bufs[bn], shz.at[idd_v.at[0]], sems[bn]).wait()
                pltpu.async_copy(h_hbm.at[ids_v.at[kn]], bufs[bn], semg[bn])

            @pl.when(k < NCHUNK)
            def _():
                pltpu.make_async_copy(
                    h_hbm.at[ids_v.at[k]], bufs[u], semg[u]).wait()
                pltpu.async_copy(bufs[u], shz.at[idd_v.at[k]], sems[u],
                                 add=True)
        return 0
    lax.fori_loop(0, (NCHUNK + 3) // 4, edge_group, 0)

    for u in range(4):
        pltpu.make_async_copy(bufs[u], shz.at[idd_v.at[0]], sems[u]).wait()

    plsc.subcore_barrier()

    base = sid * ZROWS_PER_TILE
    pltpu.sync_copy(shz.at[pl.ds(base, ZROWS_PER_TILE)],
                    zp_out.at[cid].at[pl.ds(base, ZROWS_PER_TILE)])


def _mm_body(x_ref, w_ref, degs_ref, h_ref):
    deg = jnp.sum(degs_ref[:, :N], axis=0)
    ns = lax.rsqrt(jnp.maximum(deg, 1.0))
    wsum = w_ref[0] + w_ref[1] + w_ref[2]
    hv = jnp.dot(x_ref[...] * ns[:, None], wsum,
                 preferred_element_type=jnp.float32)
    h_ref[...] = jnp.concatenate(
        [hv, jnp.zeros((SP_H - N, FOUT), jnp.float32)], axis=0)


def _fin_body(zp_ref, degd_ref, z_ref):
    degd = jnp.sum(degd_ref[:, :N], axis=0)
    nd = lax.rsqrt(jnp.maximum(degd, 1.0))
    z_ref[...] = (zp_ref[0, :N, :] + zp_ref[1, :N, :]) * nd[:, None]


_mm_call = pl.pallas_call(
    _mm_body, out_shape=jax.ShapeDtypeStruct((SP_H, FOUT), jnp.float32))
_fin_call = pl.pallas_call(
    _fin_body, out_shape=jax.ShapeDtypeStruct((N, FOUT), jnp.float32))


@jax.jit
def kernel(edge_index, X, W):
    ei = edge_index.astype(jnp.int32)
    ei3 = jnp.pad(ei, ((0, 0), (0, EPAD - E)),
                  constant_values=DUMP).reshape(2, NW, NCHUNK, CHUNK)

    degs, degd = _deg_kernel(ei3[0], ei3[1])
    h = _mm_call(X, W, degs)
    zp = _msg_kernel(h, ei3[0], ei3[1])
    return _fin_call(zp, degd)


# speedup vs baseline: 1.1231x; 1.1231x over previous
import functools

import jax
import jax.numpy as jnp
from jax import lax
from jax.experimental import pallas as pl
from jax.experimental.pallas import tpu as pltpu
from jax.experimental.pallas import tpu_sc as plsc

N = 10000
E = 320000
FIN = 128
FOUT = 32

NC = 2
NS = 16
NW = NC * NS
CHUNK = 128
NCHUNK = 79
EPW = NCHUNK * CHUNK
EPAD = EPW * NW

DUMP = 10008
SP_H = 10016
SP_Z = NCHUNK * CHUNK
ZROWS_PER_TILE = SP_Z // NS
SP_D = 10240

_mesh = plsc.VectorSubcoreMesh(core_axis_name="c", subcore_axis_name="s")


def _zeros16():
    return jnp.zeros((16,), jnp.float32)


def _ones16():
    return jnp.ones((16,), jnp.float32)


@functools.partial(
    pl.kernel,
    out_type=[
        jax.ShapeDtypeStruct((NW, SP_D), jnp.float32),
        jax.ShapeDtypeStruct((NW, SP_D), jnp.float32),
    ],
    mesh=_mesh,
    compiler_params=pltpu.CompilerParams(needs_layout_passes=False),
    scratch_types=[
        pltpu.VMEM((NCHUNK, CHUNK), jnp.int32),
        pltpu.VMEM((NCHUNK, CHUNK), jnp.int32),
        pltpu.VMEM((SP_D,), jnp.float32),
        pltpu.VMEM((SP_D,), jnp.float32),
    ],
)
def _deg_kernel(src_hbm, dst_hbm, degs_out, degd_out, ids_v, idd_v, hls, hld):
    cid = lax.axis_index("c")
    sid = lax.axis_index("s")
    w = cid * NS + sid

    pltpu.sync_copy(src_hbm.at[w], ids_v)
    pltpu.sync_copy(dst_hbm.at[w], idd_v)

    def zero_row(i, _):
        z = _zeros16()
        hls[pl.ds(i * 16, 16)] = z
        hld[pl.ds(i * 16, 16)] = z
        return 0
    lax.fori_loop(0, SP_D // 16, zero_row, 0)

    def accum(i, _):
        vs = ids_v[i // 8, pl.ds((i % 8) * 16, 16)]
        plsc.addupdate_scatter(hls, [vs], _ones16())
        vd = idd_v[i // 8, pl.ds((i % 8) * 16, 16)]
        plsc.addupdate_scatter(hld, [vd], _ones16())
        return 0
    lax.fori_loop(0, EPW // 16, accum, 0)

    pltpu.sync_copy(hls, degs_out.at[w])
    pltpu.sync_copy(hld, degd_out.at[w])


@functools.partial(
    pl.kernel,
    out_type=jax.ShapeDtypeStruct((NC, SP_Z, FOUT), jnp.float32),
    mesh=_mesh,
    compiler_params=pltpu.CompilerParams(use_tc_tiling_on_sc=False),
    scratch_types=[
        pltpu.VMEM((NCHUNK, CHUNK), jnp.int32),
        pltpu.VMEM((NCHUNK, CHUNK), jnp.int32),
        [pltpu.VMEM((CHUNK, FOUT), jnp.float32)] * 4,
        pltpu.VMEM((CHUNK, FOUT), jnp.float32),
        pltpu.VMEM_SHARED((SP_Z, FOUT), jnp.float32),
        [pltpu.SemaphoreType.DMA] * 4,
        [pltpu.SemaphoreType.DMA] * 4,
    ],
)
def _msg_kernel(h_hbm, src_hbm, dst_hbm, zp_out,
                ids_v, idd_v, bufs, zblk, shz, semg, sems):
    cid = lax.axis_index("c")
    sid = lax.axis_index("s")
    w = cid * NS + sid

    pltpu.sync_copy(src_hbm.at[w], ids_v)
    pltpu.sync_copy(dst_hbm.at[w], idd_v)

    def zero_row(i, _):
        z = _zeros16()
        zblk[i, pl.ds(0, 16)] = z
        zblk[i, pl.ds(16, 16)] = z
        return 0
    lax.fori_loop(0, CHUNK, zero_row, 0)

    for k in range((NCHUNK + NS - 1) // NS):
        j = sid + k * NS

        @pl.when(j < NCHUNK)
        def _():
            pltpu.sync_copy(zblk, shz.at[pl.ds(j * CHUNK, CHUNK)])

    plsc.subcore_barrier()

    pltpu.async_copy(h_hbm.at[ids_v.at[0]], bufs[0], semg[0])

    def edge_group(g, _):
        for u in range(4):
            k = 4 * g + u
            bn = (u + 1) % 4
            kn = k + 1

            @pl.when(kn < NCHUNK)
            def _():
                @pl.when(k - 3 >= 0)
                def _():
                    pltpu.make_async_copy(
                        bufs[bn], shz.at[idd_v.at[0]], sems[bn]).wait()
                pltpu.async_copy(h_hbm.at[ids_v.at[kn]], bufs[bn], semg[bn])

            @pl.when(k < NCHUNK)
            def _():
                pltpu.make_async_copy(
                    h_hbm.at[ids_v.at[k]], bufs[u], semg[u]).wait()
                pltpu.async_copy(bufs[u], shz.at[idd_v.at[k]], sems[u],
                                 add=True)
        return 0
    lax.fori_loop(0, (NCHUNK + 3) // 4, edge_group, 0)

    for u in range(4):
        pltpu.make_async_copy(bufs[u], shz.at[idd_v.at[0]], sems[u]).wait()

    plsc.subcore_barrier()

    base = sid * ZROWS_PER_TILE
    pltpu.sync_copy(shz.at[pl.ds(base, ZROWS_PER_TILE)],
                    zp_out.at[cid].at[pl.ds(base, ZROWS_PER_TILE)])


def _mm_body(x_ref, w_ref, degs_ref, h_ref):
    deg = jnp.sum(degs_ref[:, :N], axis=0)
    ns = lax.rsqrt(jnp.maximum(deg, 1.0))
    wsum = w_ref[0] + w_ref[1] + w_ref[2]
    hv = jnp.dot(x_ref[...] * ns[:, None], wsum,
                 preferred_element_type=jnp.float32)
    h_ref[...] = jnp.concatenate(
        [hv, jnp.zeros((SP_H - N, FOUT), jnp.float32)], axis=0)


def _fin_body(zp_ref, degd_ref, z_ref):
    degd = jnp.sum(degd_ref[:, :N], axis=0)
    nd = lax.rsqrt(jnp.maximum(degd, 1.0))
    z_ref[...] = (zp_ref[0, :N, :] + zp_ref[1, :N, :]) * nd[:, None]


_mm_call = pl.pallas_call(
    _mm_body, out_shape=jax.ShapeDtypeStruct((SP_H, FOUT), jnp.float32))
_fin_call = pl.pallas_call(
    _fin_body, out_shape=jax.ShapeDtypeStruct((N, FOUT), jnp.float32))


@jax.jit
def kernel(edge_index, X, W):
    ei = edge_index.astype(jnp.int32)
    pad = jnp.full((EPAD - E,), DUMP, jnp.int32)
    src3 = jnp.concatenate([ei[0], pad]).reshape(NW, NCHUNK, CHUNK)
    dst3 = jnp.concatenate([ei[1], pad]).reshape(NW, NCHUNK, CHUNK)

    degs, degd = _deg_kernel(src3, dst3)
    h = _mm_call(X, W, degs)
    zp = _msg_kernel(h, src3, dst3)
    return _fin_call(zp, degd)
